# baseline (speedup 1.0000x reference)
import functools

import jax
import jax.numpy as jnp
from jax import lax
from jax.experimental import pallas as pl
from jax.experimental.pallas import tpu as pltpu
from jax.experimental.pallas import tpu_sc as plsc

NUM_CHEM = 6000
NUM_DIS = 4000
HIDDEN = 128
E_POS = 160000
BATCH = 16384

NC_P = 6144
ND_P = 4096

NT = 16
CHUNK = 128
E_TILE = 10240
E_PAD = NT * E_TILE
NCHUNK = E_TILE // CHUNK

_mesh = plsc.VectorSubcoreMesh(core_axis_name="c", subcore_axis_name="s")


def _make_seg_sums(with_cnt):
    out_type = [
        jax.ShapeDtypeStruct((ND_P, HIDDEN), jnp.float32),
        jax.ShapeDtypeStruct((NC_P, HIDDEN), jnp.float32),
    ]
    scratch = [
        pltpu.VMEM((NCHUNK, CHUNK), jnp.int32),
        pltpu.VMEM((NCHUNK, CHUNK), jnp.int32),
        pltpu.VMEM((2, CHUNK, HIDDEN), jnp.float32),
        pltpu.VMEM((CHUNK,), jnp.float32),
        pltpu.VMEM((8, HIDDEN), jnp.float32),
        pltpu.VMEM_SHARED((NC_P, HIDDEN), jnp.float32),
        pltpu.VMEM_SHARED((NC_P,), jnp.float32),
    ] + [pltpu.SemaphoreType.DMA] * 5
    if with_cnt:
        out_type = out_type + [
            jax.ShapeDtypeStruct((ND_P,), jnp.float32),
            jax.ShapeDtypeStruct((NC_P,), jnp.float32),
        ]

    @functools.partial(pl.kernel, mesh=_mesh, out_type=out_type,
                       scratch_types=scratch)
    def _seg(xc_hbm, xd_hbm, tc_hbm, td_hbm, sum0_hbm, sum1_hbm, *rest):
        if with_cnt:
            cnt0_hbm, cnt1_hbm = rest[0], rest[1]
            rest = rest[2:]
        (sidx, didx, rows, ones, zbuf, acc, cacc, *sems) = rest
        gsem, ssem, csem = sems[0:2], sems[2:4], sems[4]
        cid = lax.axis_index("c")
        sid = lax.axis_index("s")

        for i in range(8):
            for j in range(HIDDEN // 16):
                zbuf[i, pl.ds(j * 16, 16)] = jnp.zeros((16,), jnp.float32)
        for j in range(CHUNK // 16):
            ones[pl.ds(j * 16, 16)] = jnp.ones((16,), jnp.float32)

        r0 = ND_P // NT
        r1 = NC_P // NT
        for b in range(r1 // 8):
            pltpu.sync_copy(zbuf, acc.at[pl.ds(sid * r1 + b * 8, 8)])
        if with_cnt:
            for b in range(r1 // HIDDEN):
                pltpu.sync_copy(zbuf.at[0],
                                cacc.at[pl.ds(sid * r1 + b * HIDDEN, HIDDEN)])
        plsc.subcore_barrier()

        def do_rel(src3_hbm, x_hbm, dst3_hbm, acc, cacc):
            pltpu.sync_copy(src3_hbm.at[sid], sidx)
            pltpu.sync_copy(dst3_hbm.at[sid], didx)

            def gather(k, b):
                return pltpu.make_async_copy(
                    x_hbm.at[sidx.at[k]], rows.at[b], gsem[b])

            def scatter_start(k, b):
                pltpu.async_copy(rows.at[b], acc.at[didx.at[k]], ssem[b],
                                 add=True)

            def scatter_wait(k, b):
                pltpu.make_async_copy(
                    rows.at[b], acc.at[didx.at[k]], ssem[b]).wait()

            gather(0, 0).start()

            def body(j, carry):
                for b in range(2):
                    k = 2 * j + b
                    gather(k, b).wait()
                    scatter_start(k, b)
                    if with_cnt:
                        pltpu.async_copy(ones, cacc.at[didx.at[k]], csem,
                                         add=True)
                    bn = 1 - b

                    @pl.when(k + 1 < NCHUNK)
                    def _():
                        @pl.when(k >= 1)
                        def _():
                            scatter_wait(k - 1, bn)
                        gather(k + 1, bn).start()
                return carry
            lax.fori_loop(0, NCHUNK // 2, body, 0)

            for b in range(2):
                scatter_wait(NCHUNK - 2 + b, b)
            if with_cnt:
                def cbody(k, carry):
                    pltpu.make_async_copy(
                        ones, cacc.at[didx.at[0]], csem).wait()
                    return carry
                lax.fori_loop(0, NCHUNK, cbody, 0)

        @pl.when(cid == 0)
        def _():
            do_rel(tc_hbm, xc_hbm, td_hbm, acc, cacc)

        @pl.when(cid == 1)
        def _():
            do_rel(td_hbm, xd_hbm, tc_hbm, acc, cacc)

        plsc.subcore_barrier()

        @pl.when(cid == 0)
        def _():
            pltpu.sync_copy(acc.at[pl.ds(sid * r0, r0)],
                            sum0_hbm.at[pl.ds(sid * r0, r0)])
            if with_cnt:
                @pl.when(sid == 0)
                def _():
                    pltpu.sync_copy(cacc.at[pl.ds(0, ND_P)], cnt0_hbm)

        @pl.when(cid == 1)
        def _():
            pltpu.sync_copy(acc.at[pl.ds(sid * r1, r1)],
                            sum1_hbm.at[pl.ds(sid * r1, r1)])
            if with_cnt:
                @pl.when(sid == 0)
                def _():
                    pltpu.sync_copy(cacc, cnt1_hbm)

    return _seg


_seg_sums_l1 = _make_seg_sums(True)
_seg_sums_l2 = _make_seg_sums(False)


B_TILE = BATCH // 32


@functools.partial(
    pl.kernel,
    mesh=_mesh,
    out_type=[
        jax.ShapeDtypeStruct((BATCH, HIDDEN), jnp.float32),
        jax.ShapeDtypeStruct((BATCH, HIDDEN), jnp.float32),
    ],
    scratch_types=[
        pltpu.VMEM((CHUNK,), jnp.int32),
        pltpu.VMEM((CHUNK, HIDDEN), jnp.float32),
        pltpu.SemaphoreType.DMA,
    ],
)
def _pair_gather(xc_hbm, xd_hbm, cid_hbm, did_hbm, cout_hbm, dout_hbm,
                 idxv, rows, sem):
    cid = lax.axis_index("c")
    sid = lax.axis_index("s")
    wid = sid * 2 + cid
    base = wid * B_TILE

    def do_tab(ids_hbm, x_hbm, out_hbm):
        def body(k, carry):
            off = base + k * CHUNK
            pltpu.sync_copy(ids_hbm.at[pl.ds(off, CHUNK)], idxv)
            pltpu.async_copy(x_hbm.at[idxv], rows, sem).wait()
            pltpu.sync_copy(rows, out_hbm.at[pl.ds(off, CHUNK)])
            return carry
        lax.fori_loop(0, B_TILE // CHUNK, body, 0)

    do_tab(cid_hbm, xc_hbm, cout_hbm)
    do_tab(did_hbm, xd_hbm, dout_hbm)


def _combine_body(x_ref, s_ref, cnt_ref, wroot_ref, wrel_ref, b_ref, o_ref,
                  *, relu):
    scale = 1.0 / jnp.maximum(cnt_ref[...], 1.0)
    acc = jnp.dot(x_ref[...], wroot_ref[...],
                  preferred_element_type=jnp.float32)
    acc = acc + b_ref[...]
    acc = acc + jnp.dot(s_ref[...] * scale, wrel_ref[...],
                        preferred_element_type=jnp.float32)
    if relu:
        acc = jnp.maximum(acc, 0.0)
    o_ref[...] = acc


def _combine(x, s, cnt, wroot, wrel, b, relu):
    n = x.shape[0]
    blk = 512
    kern = functools.partial(_combine_body, relu=relu)
    return pl.pallas_call(
        kern,
        grid=(n // blk,),
        in_specs=[
            pl.BlockSpec((blk, HIDDEN), lambda i: (i, 0)),
            pl.BlockSpec((blk, HIDDEN), lambda i: (i, 0)),
            pl.BlockSpec((blk, 1), lambda i: (i, 0)),
            pl.BlockSpec((HIDDEN, HIDDEN), lambda i: (0, 0)),
            pl.BlockSpec((HIDDEN, HIDDEN), lambda i: (0, 0)),
            pl.BlockSpec((1, HIDDEN), lambda i: (0, 0)),
        ],
        out_specs=pl.BlockSpec((blk, HIDDEN), lambda i: (i, 0)),
        out_shape=jax.ShapeDtypeStruct((n, HIDDEN), jnp.float32),
    )(x, s, cnt, wroot, wrel, b)


def _score_body(c_ref, d_ref, w_ref, o_ref):
    cw = jnp.dot(c_ref[...], w_ref[...], preferred_element_type=jnp.float32)
    o_ref[...] = jnp.sum(cw * d_ref[...], axis=1, keepdims=True)


def _score(c, d, w):
    blk = 512
    return pl.pallas_call(
        _score_body,
        grid=(BATCH // blk,),
        in_specs=[
            pl.BlockSpec((blk, HIDDEN), lambda i: (i, 0)),
            pl.BlockSpec((blk, HIDDEN), lambda i: (i, 0)),
            pl.BlockSpec((HIDDEN, HIDDEN), lambda i: (0, 0)),
        ],
        out_specs=pl.BlockSpec((blk, 1), lambda i: (i, 0)),
        out_shape=jax.ShapeDtypeStruct((BATCH, 1), jnp.float32),
    )(c, d, w)


def kernel(chem_ids, dis_ids, train_chem, train_dis, node_emb, w_rel, w_root,
           bias, W):
    f32 = jnp.float32
    pad_e = E_PAD - E_POS
    tc_pad = jnp.concatenate(
        [train_chem.astype(jnp.int32), jnp.full((pad_e,), NUM_CHEM, jnp.int32)]
    ).reshape(NT, NCHUNK, CHUNK)
    td_pad = jnp.concatenate(
        [train_dis.astype(jnp.int32), jnp.full((pad_e,), NUM_DIS, jnp.int32)]
    ).reshape(NT, NCHUNK, CHUNK)

    xc = jnp.zeros((NC_P, HIDDEN), f32).at[:NUM_CHEM].set(node_emb[:NUM_CHEM])
    xd = jnp.zeros((ND_P, HIDDEN), f32).at[:NUM_DIS].set(node_emb[NUM_CHEM:])

    cnt0_keep = cnt1_keep = None
    for l in range(2):
        if l == 0:
            sum0, sum1, cnt0, cnt1 = _seg_sums_l1(xc, xd, tc_pad, td_pad)
            cnt0_keep, cnt1_keep = cnt0.reshape(ND_P, 1), cnt1.reshape(NC_P, 1)
        else:
            sum0, sum1 = _seg_sums_l2(xc, xd, tc_pad, td_pad)
        relu = l == 0
        b2 = bias[l].reshape(1, HIDDEN)
        xc = _combine(xc, sum1, cnt1_keep, w_root[l], w_rel[l, 1], b2, relu)
        xd = _combine(xd, sum0, cnt0_keep, w_root[l], w_rel[l, 0], b2, relu)

    c_rows, d_rows = _pair_gather(xc, xd, chem_ids.astype(jnp.int32),
                                  dis_ids.astype(jnp.int32))
    return _score(c_rows, d_rows, W)[:, 0]


# speedup vs baseline: 1.0663x; 1.0663x over previous
import functools

import jax
import jax.numpy as jnp
from jax import lax
from jax.experimental import pallas as pl
from jax.experimental.pallas import tpu as pltpu
from jax.experimental.pallas import tpu_sc as plsc

NUM_CHEM = 6000
NUM_DIS = 4000
HIDDEN = 128
E_POS = 160000
BATCH = 16384

NC_P = 6144
ND_P = 4096

NT = 16
CHUNK = 128
E_TILE = 10240
E_PAD = NT * E_TILE
NCHUNK = E_TILE // CHUNK

_mesh = plsc.VectorSubcoreMesh(core_axis_name="c", subcore_axis_name="s")


def _make_seg_sums(with_cnt):
    out_type = [
        jax.ShapeDtypeStruct((ND_P, HIDDEN), jnp.float32),
        jax.ShapeDtypeStruct((NC_P, HIDDEN), jnp.float32),
    ]
    scratch = [
        pltpu.VMEM((NCHUNK, CHUNK), jnp.int32),
        pltpu.VMEM((NCHUNK, CHUNK), jnp.int32),
        pltpu.VMEM((2, CHUNK, HIDDEN), jnp.float32),
        pltpu.VMEM((CHUNK,), jnp.float32),
        pltpu.VMEM((32, HIDDEN), jnp.float32),
        pltpu.VMEM_SHARED((NC_P, HIDDEN), jnp.float32),
        pltpu.VMEM_SHARED((NC_P,), jnp.float32),
    ] + [pltpu.SemaphoreType.DMA] * 5
    if with_cnt:
        out_type = out_type + [
            jax.ShapeDtypeStruct((ND_P,), jnp.float32),
            jax.ShapeDtypeStruct((NC_P,), jnp.float32),
        ]

    @functools.partial(pl.kernel, mesh=_mesh, out_type=out_type,
                       scratch_types=scratch)
    def _seg(xc_hbm, xd_hbm, tc_hbm, td_hbm, sum0_hbm, sum1_hbm, *rest):
        if with_cnt:
            cnt0_hbm, cnt1_hbm = rest[0], rest[1]
            rest = rest[2:]
        (sidx, didx, rows, ones, zbuf, acc, cacc, *sems) = rest
        gsem, ssem, csem = sems[0:2], sems[2:4], sems[4]
        cid = lax.axis_index("c")
        sid = lax.axis_index("s")

        for i in range(32):
            for j in range(HIDDEN // 16):
                zbuf[i, pl.ds(j * 16, 16)] = jnp.zeros((16,), jnp.float32)
        for j in range(CHUNK // 16):
            ones[pl.ds(j * 16, 16)] = jnp.ones((16,), jnp.float32)

        r0 = ND_P // NT
        r1 = NC_P // NT
        for b in range(r1 // 32):
            pltpu.sync_copy(zbuf, acc.at[pl.ds(sid * r1 + b * 32, 32)])
        if with_cnt:
            for b in range(r1 // CHUNK):
                pltpu.sync_copy(zbuf.at[0],
                                cacc.at[pl.ds(sid * r1 + b * CHUNK, CHUNK)])
        plsc.subcore_barrier()

        def do_rel(src3_hbm, x_hbm, dst3_hbm, acc, cacc):
            pltpu.sync_copy(src3_hbm.at[sid], sidx)
            pltpu.sync_copy(dst3_hbm.at[sid], didx)

            def gather(k, b):
                return pltpu.make_async_copy(
                    x_hbm.at[sidx.at[k]], rows.at[b], gsem[b])

            def scatter_sync(k, b):
                pltpu.sync_copy(rows.at[b], acc.at[didx.at[k]], add=True)
                if with_cnt:
                    pltpu.async_copy(ones, cacc.at[didx.at[k]], csem,
                                     add=True)

            gather(0, 0).start()

            def body(j, carry):
                a = 2 * j
                gather(a + 1, 1).start()
                gather(a, 0).wait()
                scatter_sync(a, 0)

                @pl.when(j < NCHUNK // 2 - 1)
                def _():
                    gather(a + 2, 0).start()

                gather(a + 1, 1).wait()
                scatter_sync(a + 1, 1)
                return carry
            lax.fori_loop(0, NCHUNK // 2, body, 0)
            if with_cnt:
                def cbody(k, carry):
                    pltpu.make_async_copy(
                        ones, cacc.at[didx.at[0]], csem).wait()
                    return carry
                lax.fori_loop(0, NCHUNK, cbody, 0)

        @pl.when(cid == 0)
        def _():
            do_rel(tc_hbm, xc_hbm, td_hbm, acc, cacc)

        @pl.when(cid == 1)
        def _():
            do_rel(td_hbm, xd_hbm, tc_hbm, acc, cacc)

        plsc.subcore_barrier()

        @pl.when(cid == 0)
        def _():
            pltpu.sync_copy(acc.at[pl.ds(sid * r0, r0)],
                            sum0_hbm.at[pl.ds(sid * r0, r0)])
            if with_cnt:
                @pl.when(sid == 0)
                def _():
                    pltpu.sync_copy(cacc.at[pl.ds(0, ND_P)], cnt0_hbm)

        @pl.when(cid == 1)
        def _():
            pltpu.sync_copy(acc.at[pl.ds(sid * r1, r1)],
                            sum1_hbm.at[pl.ds(sid * r1, r1)])
            if with_cnt:
                @pl.when(sid == 0)
                def _():
                    pltpu.sync_copy(cacc, cnt1_hbm)

    return _seg


_seg_sums_l1 = _make_seg_sums(True)
_seg_sums_l2 = _make_seg_sums(False)


B_TILE = BATCH // 32


@functools.partial(
    pl.kernel,
    mesh=_mesh,
    out_type=[
        jax.ShapeDtypeStruct((BATCH, HIDDEN), jnp.float32),
        jax.ShapeDtypeStruct((BATCH, HIDDEN), jnp.float32),
    ],
    scratch_types=[
        pltpu.VMEM((CHUNK,), jnp.int32),
        pltpu.VMEM((CHUNK, HIDDEN), jnp.float32),
        pltpu.SemaphoreType.DMA,
    ],
)
def _pair_gather(xc_hbm, xd_hbm, cid_hbm, did_hbm, cout_hbm, dout_hbm,
                 idxv, rows, sem):
    cid = lax.axis_index("c")
    sid = lax.axis_index("s")
    wid = sid * 2 + cid
    base = wid * B_TILE

    def do_tab(ids_hbm, x_hbm, out_hbm):
        def body(k, carry):
            off = base + k * CHUNK
            pltpu.sync_copy(ids_hbm.at[pl.ds(off, CHUNK)], idxv)
            pltpu.async_copy(x_hbm.at[idxv], rows, sem).wait()
            pltpu.sync_copy(rows, out_hbm.at[pl.ds(off, CHUNK)])
            return carry
        lax.fori_loop(0, B_TILE // CHUNK, body, 0)

    do_tab(cid_hbm, xc_hbm, cout_hbm)
    do_tab(did_hbm, xd_hbm, dout_hbm)


def _combine_body(x_ref, s_ref, cnt_ref, wroot_ref, wrel_ref, b_ref, o_ref,
                  *, relu):
    scale = 1.0 / jnp.maximum(cnt_ref[...], 1.0)
    acc = jnp.dot(x_ref[...], wroot_ref[...],
                  preferred_element_type=jnp.float32)
    acc = acc + b_ref[...]
    acc = acc + jnp.dot(s_ref[...] * scale, wrel_ref[...],
                        preferred_element_type=jnp.float32)
    if relu:
        acc = jnp.maximum(acc, 0.0)
    o_ref[...] = acc


def _combine(x, s, cnt, wroot, wrel, b, relu):
    n = x.shape[0]
    blk = 512
    kern = functools.partial(_combine_body, relu=relu)
    return pl.pallas_call(
        kern,
        grid=(n // blk,),
        in_specs=[
            pl.BlockSpec((blk, HIDDEN), lambda i: (i, 0)),
            pl.BlockSpec((blk, HIDDEN), lambda i: (i, 0)),
            pl.BlockSpec((blk, 1), lambda i: (i, 0)),
            pl.BlockSpec((HIDDEN, HIDDEN), lambda i: (0, 0)),
            pl.BlockSpec((HIDDEN, HIDDEN), lambda i: (0, 0)),
            pl.BlockSpec((1, HIDDEN), lambda i: (0, 0)),
        ],
        out_specs=pl.BlockSpec((blk, HIDDEN), lambda i: (i, 0)),
        out_shape=jax.ShapeDtypeStruct((n, HIDDEN), jnp.float32),
    )(x, s, cnt, wroot, wrel, b)


def _score_body(c_ref, d_ref, w_ref, o_ref):
    cw = jnp.dot(c_ref[...], w_ref[...], preferred_element_type=jnp.float32)
    o_ref[...] = jnp.sum(cw * d_ref[...], axis=1, keepdims=True)


def _score(c, d, w):
    blk = 512
    return pl.pallas_call(
        _score_body,
        grid=(BATCH // blk,),
        in_specs=[
            pl.BlockSpec((blk, HIDDEN), lambda i: (i, 0)),
            pl.BlockSpec((blk, HIDDEN), lambda i: (i, 0)),
            pl.BlockSpec((HIDDEN, HIDDEN), lambda i: (0, 0)),
        ],
        out_specs=pl.BlockSpec((blk, 1), lambda i: (i, 0)),
        out_shape=jax.ShapeDtypeStruct((BATCH, 1), jnp.float32),
    )(c, d, w)


def kernel(chem_ids, dis_ids, train_chem, train_dis, node_emb, w_rel, w_root,
           bias, W):
    f32 = jnp.float32
    pad_e = E_PAD - E_POS
    tc_pad = jnp.concatenate(
        [train_chem.astype(jnp.int32), jnp.full((pad_e,), NUM_CHEM, jnp.int32)]
    ).reshape(NT, NCHUNK, CHUNK)
    td_pad = jnp.concatenate(
        [train_dis.astype(jnp.int32), jnp.full((pad_e,), NUM_DIS, jnp.int32)]
    ).reshape(NT, NCHUNK, CHUNK)

    xc = jnp.zeros((NC_P, HIDDEN), f32).at[:NUM_CHEM].set(node_emb[:NUM_CHEM])
    xd = jnp.zeros((ND_P, HIDDEN), f32).at[:NUM_DIS].set(node_emb[NUM_CHEM:])

    cnt0_keep = cnt1_keep = None
    for l in range(2):
        if l == 0:
            sum0, sum1, cnt0, cnt1 = _seg_sums_l1(xc, xd, tc_pad, td_pad)
            cnt0_keep, cnt1_keep = cnt0.reshape(ND_P, 1), cnt1.reshape(NC_P, 1)
        else:
            sum0, sum1 = _seg_sums_l2(xc, xd, tc_pad, td_pad)
        relu = l == 0
        b2 = bias[l].reshape(1, HIDDEN)
        xc = _combine(xc, sum1, cnt1_keep, w_root[l], w_rel[l, 1], b2, relu)
        xd = _combine(xd, sum0, cnt0_keep, w_root[l], w_rel[l, 0], b2, relu)

    c_rows, d_rows = _pair_gather(xc, xd, chem_ids.astype(jnp.int32),
                                  dis_ids.astype(jnp.int32))
    return _score(c_rows, d_rows, W)[:, 0]
